# baseline (speedup 1.0000x reference)
import functools

import jax
import jax.numpy as jnp
from jax import lax
from jax.experimental import pallas as pl
from jax.experimental.pallas import tpu as pltpu
from jax.experimental.pallas import tpu_sc as plsc

_NC = 2
_NS = 16
_NW = _NC * _NS

_VOCAB = 1000000
_BATCH = 4096
_BAG = 50
_DIM = 64
_SCRATCH_W = 128

_TBLK = 128
_NFULL = _VOCAB // _TBLK
_TAIL = _VOCAB - _NFULL * _TBLK
_BLKS_PER_W = (_NFULL + _NW - 1) // _NW

_BAGS_PER_W = _BATCH // _NW
_BAGS_PER_CHUNK = 2
_CHUNK = _BAGS_PER_CHUNK * _BAG
_NCHUNKS = _BAGS_PER_W // _BAGS_PER_CHUNK

_MESH = dict(core_axis_name="c", subcore_axis_name="s",
             num_cores=_NC, num_subcores=_NS)


def _transpose_block(chunk_ref, rows_ref):
    iota = lax.broadcasted_iota(jnp.int32, (16,), 0)
    row_idx = [iota + 16 * dg for dg in range(_DIM // 16)]

    def tok_group(g, carry):
        base = g * 16
        for j in range(16):
            t = base + j
            col_idx = jnp.full((16,), 0, jnp.int32) + t
            for dg in range(_DIM // 16):
                v = plsc.load_gather(chunk_ref, [row_idx[dg], col_idx])
                rows_ref[t, pl.ds(dg * 16, 16)] = v
        return carry

    lax.fori_loop(0, _TBLK // 16, tok_group, 0)


def _transpose_sc(table_t, tail_rows):
    mesh = plsc.VectorSubcoreMesh(**_MESH)

    @functools.partial(
        pl.kernel,
        out_type=jax.ShapeDtypeStruct((_VOCAB, _SCRATCH_W), jnp.float32),
        mesh=mesh,
        scratch_types=[
            pltpu.VMEM((_DIM, _TBLK), jnp.float32),
            pltpu.VMEM((_DIM, _TBLK), jnp.float32),
            pltpu.VMEM((_TBLK, _SCRATCH_W), jnp.float32),
            pltpu.VMEM((_TBLK, _SCRATCH_W), jnp.float32),
            pltpu.VMEM((_TAIL, _SCRATCH_W), jnp.float32),
            pltpu.SemaphoreType.DMA,
            pltpu.SemaphoreType.DMA,
            pltpu.SemaphoreType.DMA,
            pltpu.SemaphoreType.DMA,
        ],
        compiler_params=pltpu.CompilerParams(needs_layout_passes=False),
    )
    def k(tt_hbm, tail_hbm, out_hbm, chunk_a, chunk_b, rows_a, rows_b,
          tail_v, isem_a, isem_b, osem_a, osem_b):
        wid = lax.axis_index("s") * _NC + lax.axis_index("c")

        def blk(i):
            return i * _NW + wid

        def fetch(b, buf, sem):
            pltpu.async_copy(tt_hbm.at[:, pl.ds(b * _TBLK, _TBLK)], buf, sem)

        def put(b, buf, sem):
            pltpu.async_copy(buf, out_hbm.at[pl.ds(b * _TBLK, _TBLK)], sem)

        @pl.when(blk(0) < _NFULL)
        def _():
            fetch(blk(0), chunk_a, isem_a)

        @pl.when(blk(1) < _NFULL)
        def _():
            fetch(blk(1), chunk_b, isem_b)

        def step(p, carry):
            ba, bb = blk(2 * p), blk(2 * p + 1)

            @pl.when(ba < _NFULL)
            def _():
                pltpu.make_async_copy(
                    tt_hbm.at[:, pl.ds(ba * _TBLK, _TBLK)], chunk_a,
                    isem_a).wait()
                @pl.when(p > 0)
                def _():
                    pltpu.make_async_copy(
                        rows_a, out_hbm.at[pl.ds(0, _TBLK)], osem_a).wait()
                _transpose_block(chunk_a, rows_a)

                @pl.when(blk(2 * p + 2) < _NFULL)
                def _():
                    fetch(blk(2 * p + 2), chunk_a, isem_a)
                put(ba, rows_a, osem_a)

            @pl.when(bb < _NFULL)
            def _():
                pltpu.make_async_copy(
                    tt_hbm.at[:, pl.ds(bb * _TBLK, _TBLK)], chunk_b,
                    isem_b).wait()

                @pl.when(p > 0)
                def _():
                    pltpu.make_async_copy(
                        rows_b, out_hbm.at[pl.ds(0, _TBLK)], osem_b).wait()
                _transpose_block(chunk_b, rows_b)

                @pl.when(blk(2 * p + 3) < _NFULL)
                def _():
                    fetch(blk(2 * p + 3), chunk_b, isem_b)
                put(bb, rows_b, osem_b)

            return carry

        lax.fori_loop(0, (_BLKS_PER_W + 1) // 2, step, 0)

        n_mine = (_NFULL - wid + _NW - 1) // _NW

        @pl.when(n_mine >= 1)
        def _():
            pltpu.make_async_copy(
                rows_a, out_hbm.at[pl.ds(0, _TBLK)], osem_a).wait()

        @pl.when(n_mine >= 2)
        def _():
            pltpu.make_async_copy(
                rows_b, out_hbm.at[pl.ds(0, _TBLK)], osem_b).wait()

        @pl.when(wid == 0)
        def _():
            pltpu.sync_copy(tail_hbm, tail_v)
            pltpu.sync_copy(tail_v, out_hbm.at[pl.ds(_NFULL * _TBLK, _TAIL)])

    return k(table_t, tail_rows)


def _bag_reduce(rows_ref, feat_ref, first_bag):
    for b in range(_BAGS_PER_CHUNK):
        base = b * _BAG
        for cc in range(_DIM // 16):
            col = pl.ds(cc * 16, 16)
            acc = rows_ref[base, col]
            for r in range(1, _BAG):
                acc = acc + rows_ref[base + r, col]
            feat_ref[first_bag + b, col] = acc


def _embedding_bag_sc(bow3, scratch):
    mesh = plsc.VectorSubcoreMesh(**_MESH)

    @functools.partial(
        pl.kernel,
        out_type=jax.ShapeDtypeStruct((_BATCH, _DIM), jnp.float32),
        mesh=mesh,
        scratch_types=[
            pltpu.VMEM((_NCHUNKS, _CHUNK), jnp.int32),
            pltpu.VMEM((_CHUNK, _SCRATCH_W), jnp.float32),
            pltpu.VMEM((_CHUNK, _SCRATCH_W), jnp.float32),
            pltpu.VMEM((_BAGS_PER_W, _DIM), jnp.float32),
            pltpu.SemaphoreType.DMA,
            pltpu.SemaphoreType.DMA,
        ],
    )
    def k(bow_hbm, table_hbm, out_hbm, idx_v, rows_a, rows_b, feat_v,
          sem_a, sem_b):
        wid = lax.axis_index("s") * _NC + lax.axis_index("c")
        pltpu.sync_copy(bow_hbm.at[wid], idx_v)
        pltpu.async_copy(table_hbm.at[idx_v.at[0]], rows_a, sem_a)

        def step(i, carry):
            pltpu.make_async_copy(table_hbm.at[idx_v.at[2 * i]],
                                  rows_a, sem_a).wait()
            pltpu.async_copy(table_hbm.at[idx_v.at[2 * i + 1]], rows_b, sem_b)
            _bag_reduce(rows_a, feat_v, 4 * i)

            @pl.when(i < _NCHUNKS // 2 - 1)
            def _():
                pltpu.async_copy(table_hbm.at[idx_v.at[2 * i + 2]],
                                 rows_a, sem_a)

            pltpu.make_async_copy(table_hbm.at[idx_v.at[2 * i + 1]],
                                  rows_b, sem_b).wait()
            _bag_reduce(rows_b, feat_v, 4 * i + 2)
            return carry

        lax.fori_loop(0, _NCHUNKS // 2, step, 0)
        pltpu.sync_copy(feat_v, out_hbm.at[pl.ds(wid * _BAGS_PER_W,
                                                 _BAGS_PER_W)])

    return k(bow3, scratch)


def _classifier_tc(features, W, b2):
    def body(f_ref, w_ref, b_ref, o_ref):
        f = f_ref[...]
        w = w_ref[...]
        logits = lax.dot_general(f, w, (((1,), (1,)), ((), ())),
                                 preferred_element_type=jnp.float32)
        logits = logits + b_ref[...]
        m = jnp.max(logits, axis=1, keepdims=True)
        e = jnp.exp(logits - m)
        s = jnp.sum(e, axis=1, keepdims=True)
        o_ref[...] = logits - m - jnp.log(s)

    return pl.pallas_call(
        body,
        out_shape=jax.ShapeDtypeStruct((_BATCH, W.shape[0]), jnp.float32),
    )(features, W, b2)


@jax.jit
def kernel(bow, emb_weight, W, b):
    table_t = emb_weight.T
    tail_rows = jnp.pad(emb_weight[_NFULL * _TBLK:],
                        ((0, 0), (0, _SCRATCH_W - _DIM)))
    scratch = _transpose_sc(table_t, tail_rows)
    bow3 = bow.reshape(_NW, _NCHUNKS, _CHUNK)
    features = _embedding_bag_sc(bow3, scratch)
    return _classifier_tc(features, W, b.reshape(1, -1))


# speedup vs baseline: 1.2264x; 1.2264x over previous
import functools

import jax
import jax.numpy as jnp
from jax import lax
from jax.experimental import pallas as pl
from jax.experimental.pallas import tpu as pltpu
from jax.experimental.pallas import tpu_sc as plsc

_NC = 2
_NS = 16
_NW = _NC * _NS

_VOCAB = 1000000
_BATCH = 4096
_BAG = 50
_DIM = 64
_SCRATCH_W = 128

_TBLK = 512
_TGRID = (_VOCAB + _TBLK - 1) // _TBLK

_BAGS_PER_W = _BATCH // _NW
_BAGS_PER_CHUNK = 2
_CHUNK = _BAGS_PER_CHUNK * _BAG
_NCHUNKS = _BAGS_PER_W // _BAGS_PER_CHUNK

_MESH = dict(core_axis_name="c", subcore_axis_name="s",
             num_cores=_NC, num_subcores=_NS)


def _transpose_tc(table_t):
    def body(t_ref, o_ref):
        x = t_ref[...]
        o_ref[:, 0:_DIM] = x.T
        o_ref[:, _DIM:_SCRATCH_W] = jnp.zeros(
            (_TBLK, _SCRATCH_W - _DIM), jnp.float32)

    return pl.pallas_call(
        body,
        grid=(_TGRID,),
        in_specs=[pl.BlockSpec((_DIM, _TBLK), lambda i: (0, i))],
        out_specs=pl.BlockSpec((_TBLK, _SCRATCH_W), lambda i: (i, 0)),
        out_shape=jax.ShapeDtypeStruct((_VOCAB, _SCRATCH_W), jnp.float32),
    )(table_t)


def _bag_reduce(rows_ref, feat_ref, first_bag):
    for b in range(_BAGS_PER_CHUNK):
        base = b * _BAG
        for cc in range(_DIM // 16):
            col = pl.ds(cc * 16, 16)
            acc = rows_ref[base, col]
            for r in range(1, _BAG):
                acc = acc + rows_ref[base + r, col]
            feat_ref[first_bag + b, col] = acc


def _embedding_bag_sc(bow3, scratch):
    mesh = plsc.VectorSubcoreMesh(**_MESH)

    @functools.partial(
        pl.kernel,
        out_type=jax.ShapeDtypeStruct((_BATCH, _DIM), jnp.float32),
        mesh=mesh,
        scratch_types=[
            pltpu.VMEM((_NCHUNKS, _CHUNK), jnp.int32),
            pltpu.VMEM((_CHUNK, _SCRATCH_W), jnp.float32),
            pltpu.VMEM((_CHUNK, _SCRATCH_W), jnp.float32),
            pltpu.VMEM((_BAGS_PER_W, _DIM), jnp.float32),
            pltpu.SemaphoreType.DMA,
            pltpu.SemaphoreType.DMA,
        ],
    )
    def k(bow_hbm, table_hbm, out_hbm, idx_v, rows_a, rows_b, feat_v,
          sem_a, sem_b):
        wid = lax.axis_index("s") * _NC + lax.axis_index("c")
        pltpu.sync_copy(bow_hbm.at[wid], idx_v)
        pltpu.async_copy(table_hbm.at[idx_v.at[0]], rows_a, sem_a)

        def step(i, carry):
            pltpu.make_async_copy(table_hbm.at[idx_v.at[2 * i]],
                                  rows_a, sem_a).wait()
            pltpu.async_copy(table_hbm.at[idx_v.at[2 * i + 1]], rows_b, sem_b)
            _bag_reduce(rows_a, feat_v, 4 * i)

            @pl.when(i < _NCHUNKS // 2 - 1)
            def _():
                pltpu.async_copy(table_hbm.at[idx_v.at[2 * i + 2]],
                                 rows_a, sem_a)

            pltpu.make_async_copy(table_hbm.at[idx_v.at[2 * i + 1]],
                                  rows_b, sem_b).wait()
            _bag_reduce(rows_b, feat_v, 4 * i + 2)
            return carry

        lax.fori_loop(0, _NCHUNKS // 2, step, 0)
        pltpu.sync_copy(feat_v, out_hbm.at[pl.ds(wid * _BAGS_PER_W,
                                                 _BAGS_PER_W)])

    return k(bow3, scratch)


def _classifier_tc(features, W, b2):
    def body(f_ref, w_ref, b_ref, o_ref):
        f = f_ref[...]
        w = w_ref[...]
        logits = lax.dot_general(f, w, (((1,), (1,)), ((), ())),
                                 preferred_element_type=jnp.float32)
        logits = logits + b_ref[...]
        m = jnp.max(logits, axis=1, keepdims=True)
        e = jnp.exp(logits - m)
        s = jnp.sum(e, axis=1, keepdims=True)
        o_ref[...] = logits - m - jnp.log(s)

    return pl.pallas_call(
        body,
        out_shape=jax.ShapeDtypeStruct((_BATCH, W.shape[0]), jnp.float32),
    )(features, W, b2)


@jax.jit
def kernel(bow, emb_weight, W, b):
    scratch = _transpose_tc(emb_weight.T)
    bow3 = bow.reshape(_NW, _NCHUNKS, _CHUNK)
    features = _embedding_bag_sc(bow3, scratch)
    return _classifier_tc(features, W, b.reshape(1, -1))


# speedup vs baseline: 2.6959x; 2.1983x over previous
import functools

import jax
import jax.numpy as jnp
from jax import lax
from jax.experimental import pallas as pl
from jax.experimental.pallas import tpu as pltpu
from jax.experimental.pallas import tpu_sc as plsc

_NC = 2
_NS = 16
_NW = _NC * _NS

_VOCAB = 1000000
_BATCH = 4096
_BAG = 50
_DIM = 64
_SCRATCH_W = 128

_TBLK = 2048
_TGRID = (_VOCAB + _TBLK - 1) // _TBLK

_BAGS_PER_W = _BATCH // _NW
_BAGS_PER_CHUNK = 2
_CHUNK = _BAGS_PER_CHUNK * _BAG
_NCHUNKS = _BAGS_PER_W // _BAGS_PER_CHUNK

_MESH = dict(core_axis_name="c", subcore_axis_name="s",
             num_cores=_NC, num_subcores=_NS)


def _transpose_tc(table_t, eye):
    def body(t_ref, e_ref, o_ref):
        x = t_ref[...]
        o_ref[:, 0:_DIM] = lax.dot_general(
            x, e_ref[...], (((0,), (0,)), ((), ())),
            preferred_element_type=jnp.float32)

    return pl.pallas_call(
        body,
        grid=(_TGRID,),
        in_specs=[pl.BlockSpec((_DIM, _TBLK), lambda i: (0, i)),
                  pl.BlockSpec((_DIM, _DIM), lambda i: (0, 0))],
        out_specs=pl.BlockSpec((_TBLK, _SCRATCH_W), lambda i: (i, 0)),
        out_shape=jax.ShapeDtypeStruct((_VOCAB, _SCRATCH_W), jnp.float32),
    )(table_t, eye)


def _bag_reduce(rows_ref, feat_ref, first_bag):
    for b in range(_BAGS_PER_CHUNK):
        base = b * _BAG
        for cc in range(_DIM // 16):
            col = pl.ds(cc * 16, 16)
            acc = rows_ref[base, col]
            for r in range(1, _BAG):
                acc = acc + rows_ref[base + r, col]
            feat_ref[first_bag + b, col] = acc


def _embedding_bag_sc(bow3, scratch):
    mesh = plsc.VectorSubcoreMesh(**_MESH)

    @functools.partial(
        pl.kernel,
        out_type=jax.ShapeDtypeStruct((_BATCH, _DIM), jnp.float32),
        mesh=mesh,
        scratch_types=[
            pltpu.VMEM((_NCHUNKS, _CHUNK), jnp.int32),
            pltpu.VMEM((_CHUNK, _SCRATCH_W), jnp.float32),
            pltpu.VMEM((_CHUNK, _SCRATCH_W), jnp.float32),
            pltpu.VMEM((_BAGS_PER_W, _DIM), jnp.float32),
            pltpu.SemaphoreType.DMA,
            pltpu.SemaphoreType.DMA,
        ],
    )
    def k(bow_hbm, table_hbm, out_hbm, idx_v, rows_a, rows_b, feat_v,
          sem_a, sem_b):
        wid = lax.axis_index("s") * _NC + lax.axis_index("c")
        pltpu.sync_copy(bow_hbm.at[wid], idx_v)
        pltpu.async_copy(table_hbm.at[idx_v.at[0]], rows_a, sem_a)

        def step(i, carry):
            pltpu.make_async_copy(table_hbm.at[idx_v.at[2 * i]],
                                  rows_a, sem_a).wait()
            pltpu.async_copy(table_hbm.at[idx_v.at[2 * i + 1]], rows_b, sem_b)
            _bag_reduce(rows_a, feat_v, 4 * i)

            @pl.when(i < _NCHUNKS // 2 - 1)
            def _():
                pltpu.async_copy(table_hbm.at[idx_v.at[2 * i + 2]],
                                 rows_a, sem_a)

            pltpu.make_async_copy(table_hbm.at[idx_v.at[2 * i + 1]],
                                  rows_b, sem_b).wait()
            _bag_reduce(rows_b, feat_v, 4 * i + 2)
            return carry

        lax.fori_loop(0, _NCHUNKS // 2, step, 0)
        pltpu.sync_copy(feat_v, out_hbm.at[pl.ds(wid * _BAGS_PER_W,
                                                 _BAGS_PER_W)])

    return k(bow3, scratch)


def _classifier_tc(features, W, b2):
    def body(f_ref, w_ref, b_ref, o_ref):
        f = f_ref[...]
        w = w_ref[...]
        logits = lax.dot_general(f, w, (((1,), (1,)), ((), ())),
                                 preferred_element_type=jnp.float32)
        logits = logits + b_ref[...]
        m = jnp.max(logits, axis=1, keepdims=True)
        e = jnp.exp(logits - m)
        s = jnp.sum(e, axis=1, keepdims=True)
        o_ref[...] = logits - m - jnp.log(s)

    return pl.pallas_call(
        body,
        out_shape=jax.ShapeDtypeStruct((_BATCH, W.shape[0]), jnp.float32),
    )(features, W, b2)


@jax.jit
def kernel(bow, emb_weight, W, b):
    scratch = _transpose_tc(emb_weight.T,
                            jnp.eye(_DIM, dtype=jnp.float32))
    bow3 = bow.reshape(_NW, _NCHUNKS, _CHUNK)
    features = _embedding_bag_sc(bow3, scratch)
    return _classifier_tc(features, W, b.reshape(1, -1))
